# baseline (speedup 1.0000x reference)
import functools

import numpy as np
import jax
import jax.numpy as jnp
from jax import lax
from jax.experimental import pallas as pl
from jax.experimental.pallas import tpu as pltpu
from jax.experimental.pallas import tpu_sc as plsc

_B = 4
_T = 16
_FRAME = 224 * 224 * 3
_SEQ = _T * _FRAME
_CHUNK = 114688
_NCHUNK = _SEQ // _CHUNK

_TMPL = np.concatenate(
    [np.ones(_FRAME, np.bool_), np.zeros(_FRAME, np.bool_)])


def _tp_body(start_ref, end_ref, tp_ref):
    c = pl.program_id(0)
    idx = c * _CHUNK + lax.broadcasted_iota(jnp.int32, (_B, _CHUNK), 1)
    row = lax.broadcasted_iota(jnp.int32, (_B, _CHUNK), 0)

    def per_row(vals_ref):
        v0, v1, v2, v3 = vals_ref[0], vals_ref[1], vals_ref[2], vals_ref[3]
        return jnp.where(row == 0, v0,
               jnp.where(row == 1, v1,
               jnp.where(row == 2, v2, v3)))

    s = per_row(start_ref)
    e = per_row(end_ref)
    tp_ref[...] = jnp.where(idx < (e - s), s + idx, _SEQ)


@functools.cache
def _make_masks_sc():
    mesh = plsc.VectorSubcoreMesh(core_axis_name="c", subcore_axis_name="s")
    return functools.partial(
        pl.kernel,
        mesh=mesh,
        out_type=[
            jax.ShapeDtypeStruct((_B, _SEQ), jnp.bool_),
            jax.ShapeDtypeStruct((_B, _SEQ), jnp.bool_),
        ],
        scratch_types=[pltpu.VMEM((16,), jnp.int32)],
    )(_masks_sc_body)


def _masks_sc_body(tmpl_hbm, se_hbm, cm_out, tm_out, se_v):
    wid = lax.axis_index("s") * 2 + lax.axis_index("c")
    pltpu.sync_copy(se_hbm, se_v)
    sev = se_v[...]
    sfs = [sev[b] for b in range(_B)]
    efs = [sev[_B + b] for b in range(_B)]

    def sel(vals, b):
        return jnp.where(b == 0, vals[0],
               jnp.where(b == 1, vals[1],
               jnp.where(b == 2, vals[2], vals[3])))

    def run_jobs(wid2, out_ref, invert):
        for i in range(4):
            job = wid2 * 4 + i
            b = job // 16
            f = job % 16
            in_mask = (f >= sel(sfs, b)) & (f < sel(efs, b))
            want_true = in_mask != invert
            off = jnp.where(want_true, 0, _FRAME)
            pltpu.sync_copy(tmpl_hbm.at[pl.ds(off, _FRAME)],
                            out_ref.at[b, pl.ds(f * _FRAME, _FRAME)])

    @pl.when(wid < 16)
    def _():
        run_jobs(wid, tm_out, False)

    @pl.when(wid >= 16)
    def _():
        run_jobs(wid - 16, cm_out, True)


def kernel(batch_size, num_frames, frame_size, scales, rand_start):
    num_mask = jnp.clip((scales * _T).astype(jnp.int32), 1, _T - 2)
    max_start = jnp.clip(_T - num_mask - 1, 1, None)
    start_frames = (rand_start * max_start.astype(jnp.float32) + 1.0).astype(jnp.int32)
    start_pos = start_frames * _FRAME
    end_pos = jnp.minimum((start_frames + num_mask) * _FRAME, _SEQ)

    tp = pl.pallas_call(
        _tp_body,
        grid=(_NCHUNK,),
        in_specs=[
            pl.BlockSpec(memory_space=pltpu.SMEM),
            pl.BlockSpec(memory_space=pltpu.SMEM),
        ],
        out_specs=pl.BlockSpec((_B, _CHUNK), lambda c: (0, c)),
        out_shape=jax.ShapeDtypeStruct((_B, _SEQ), jnp.int32),
    )(start_pos, end_pos)

    end_frames = jnp.minimum(start_frames + num_mask, _T)
    se = jnp.zeros((16,), jnp.int32)
    se = se.at[0:4].set(start_frames).at[4:8].set(end_frames)
    cm, tm = _make_masks_sc()(jnp.asarray(_TMPL), se)
    return (cm, tm, tp)


# speedup vs baseline: 18.7530x; 18.7530x over previous
import functools

import jax
import jax.numpy as jnp
from jax import lax
from jax.experimental import pallas as pl
from jax.experimental.pallas import tpu as pltpu
from jax.experimental.pallas import tpu_sc as plsc

_B = 4
_T = 16
_FRAME = 224 * 224 * 3
_SEQ = _T * _FRAME
_CHUNK = 114688
_NCHUNK = _SEQ // _CHUNK
_VC = _FRAME // 4


def _masks_body(start_ref, end_ref, cm_ref, tm_ref):
    c = pl.program_id(0)
    idx = c * _CHUNK + lax.broadcasted_iota(jnp.int32, (_B, _CHUNK), 1)
    row = lax.broadcasted_iota(jnp.int32, (_B, _CHUNK), 0)

    def per_row(vals_ref):
        v0, v1, v2, v3 = vals_ref[0], vals_ref[1], vals_ref[2], vals_ref[3]
        return jnp.where(row == 0, v0,
               jnp.where(row == 1, v1,
               jnp.where(row == 2, v2, v3)))

    s = per_row(start_ref)
    e = per_row(end_ref)
    tm = (idx >= s) & (idx < e)
    tm_ref[...] = tm
    cm_ref[...] = ~tm


@functools.cache
def _make_tp_sc():
    mesh = plsc.VectorSubcoreMesh(core_axis_name="c", subcore_axis_name="s")
    return functools.partial(
        pl.kernel,
        mesh=mesh,
        out_type=jax.ShapeDtypeStruct((_B, _SEQ), jnp.int32),
        scratch_types=[
            pltpu.VMEM((16,), jnp.int32),
            pltpu.VMEM((_VC,), jnp.int32),
        ],
    )(_tp_sc_body)


def _tp_sc_body(se_hbm, tp_out, se_v, buf):
    wid = lax.axis_index("s") * 2 + lax.axis_index("c")
    pltpu.sync_copy(se_hbm, se_v)
    sev = se_v[...]
    sps = [sev[b] for b in range(_B)]
    ls = [sev[_B + b] for b in range(_B)]

    def sel(vals, b):
        return jnp.where(b == 0, vals[0],
               jnp.where(b == 1, vals[1],
               jnp.where(b == 2, vals[2], vals[3])))

    b = wid // 8
    r8 = wid % 8
    sp = sel(sps, b)
    ln = sel(ls, b)
    lane16 = lax.iota(jnp.int32, 16)

    col0 = r8 * (2 * _FRAME)
    for k in range(8):
        base = col0 + k * _VC

        @pl.when(base < ln)
        def _(base=base):
            def gen(i, v):
                buf[pl.ds(i * 16, 16)] = v
                return v + 16
            lax.fori_loop(0, _VC // 16, gen, sp + base + lane16)

        @pl.when(base >= ln)
        def _():
            def gen(i, v):
                buf[pl.ds(i * 16, 16)] = v
                return v
            lax.fori_loop(0, _VC // 16, gen, jnp.full((16,), _SEQ, jnp.int32))

        pltpu.sync_copy(buf, tp_out.at[b, pl.ds(base, _VC)])


def kernel(batch_size, num_frames, frame_size, scales, rand_start):
    num_mask = jnp.clip((scales * _T).astype(jnp.int32), 1, _T - 2)
    max_start = jnp.clip(_T - num_mask - 1, 1, None)
    start_frames = (rand_start * max_start.astype(jnp.float32) + 1.0).astype(jnp.int32)
    start_pos = start_frames * _FRAME
    end_pos = jnp.minimum((start_frames + num_mask) * _FRAME, _SEQ)

    cm, tm = pl.pallas_call(
        _masks_body,
        grid=(_NCHUNK,),
        in_specs=[
            pl.BlockSpec(memory_space=pltpu.SMEM),
            pl.BlockSpec(memory_space=pltpu.SMEM),
        ],
        out_specs=[
            pl.BlockSpec((_B, _CHUNK), lambda c: (0, c)),
            pl.BlockSpec((_B, _CHUNK), lambda c: (0, c)),
        ],
        out_shape=[
            jax.ShapeDtypeStruct((_B, _SEQ), jnp.bool_),
            jax.ShapeDtypeStruct((_B, _SEQ), jnp.bool_),
        ],
    )(start_pos, end_pos)

    se = jnp.zeros((16,), jnp.int32)
    se = se.at[0:4].set(start_pos).at[4:8].set(end_pos - start_pos)
    tp = _make_tp_sc()(se)
    return (cm, tm, tp)


# speedup vs baseline: 26.6381x; 1.4205x over previous
import functools

import jax
import jax.numpy as jnp
from jax import lax
from jax.experimental import pallas as pl
from jax.experimental.pallas import tpu as pltpu
from jax.experimental.pallas import tpu_sc as plsc

_B = 4
_T = 16
_FRAME = 224 * 224 * 3
_SEQ = _T * _FRAME
_CHUNK = 114688
_NCHUNK = _SEQ // _CHUNK
_VC = _FRAME // 4


def _masks_body(start_ref, end_ref, cm_ref, tm_ref):
    c = pl.program_id(0)
    idx = c * _CHUNK + lax.broadcasted_iota(jnp.int32, (_B, _CHUNK), 1)
    row = lax.broadcasted_iota(jnp.int32, (_B, _CHUNK), 0)

    def per_row(vals_ref):
        v0, v1, v2, v3 = vals_ref[0], vals_ref[1], vals_ref[2], vals_ref[3]
        return jnp.where(row == 0, v0,
               jnp.where(row == 1, v1,
               jnp.where(row == 2, v2, v3)))

    s = per_row(start_ref)
    e = per_row(end_ref)
    tm = (idx >= s) & (idx < e)
    tm_ref[...] = tm
    cm_ref[...] = ~tm


@functools.cache
def _make_tp_sc():
    mesh = plsc.VectorSubcoreMesh(core_axis_name="c", subcore_axis_name="s")
    return functools.partial(
        pl.kernel,
        mesh=mesh,
        out_type=jax.ShapeDtypeStruct((_B, _SEQ), jnp.int32),
        scratch_types=[
            pltpu.VMEM((16,), jnp.int32),
            pltpu.VMEM((_VC,), jnp.int32),
        ],
    )(_tp_sc_body)


def _tp_sc_body(se_hbm, tp_out, se_v, buf):
    wid = lax.axis_index("s") * 2 + lax.axis_index("c")
    pltpu.sync_copy(se_hbm, se_v)
    sev = se_v[...]
    sps = [sev[b] for b in range(_B)]
    ls = [sev[_B + b] for b in range(_B)]

    def sel(vals, b):
        return jnp.where(b == 0, vals[0],
               jnp.where(b == 1, vals[1],
               jnp.where(b == 2, vals[2], vals[3])))

    b = wid // 8
    r8 = wid % 8
    sp = sel(sps, b)
    ln = sel(ls, b)
    lane16 = lax.iota(jnp.int32, 16)

    col0 = r8 * (2 * _FRAME)
    for k in range(8):
        base = col0 + k * _VC

        @pl.when(base < ln)
        def _(base=base):
            def gen(i, v):
                for u in range(8):
                    buf[pl.ds(i * 128 + u * 16, 16)] = v + u * 16
                return v + 128
            lax.fori_loop(0, _VC // 128, gen, sp + base + lane16)

        @pl.when(base >= ln)
        def _():
            cv = jnp.full((16,), _SEQ, jnp.int32)

            def gen(i, v):
                for u in range(8):
                    buf[pl.ds(i * 128 + u * 16, 16)] = v
                return v
            lax.fori_loop(0, _VC // 128, gen, cv)

        pltpu.sync_copy(buf, tp_out.at[b, pl.ds(base, _VC)])


def kernel(batch_size, num_frames, frame_size, scales, rand_start):
    num_mask = jnp.clip((scales * _T).astype(jnp.int32), 1, _T - 2)
    max_start = jnp.clip(_T - num_mask - 1, 1, None)
    start_frames = (rand_start * max_start.astype(jnp.float32) + 1.0).astype(jnp.int32)
    start_pos = start_frames * _FRAME
    end_pos = jnp.minimum((start_frames + num_mask) * _FRAME, _SEQ)

    cm, tm = pl.pallas_call(
        _masks_body,
        grid=(_NCHUNK,),
        in_specs=[
            pl.BlockSpec(memory_space=pltpu.SMEM),
            pl.BlockSpec(memory_space=pltpu.SMEM),
        ],
        out_specs=[
            pl.BlockSpec((_B, _CHUNK), lambda c: (0, c)),
            pl.BlockSpec((_B, _CHUNK), lambda c: (0, c)),
        ],
        out_shape=[
            jax.ShapeDtypeStruct((_B, _SEQ), jnp.bool_),
            jax.ShapeDtypeStruct((_B, _SEQ), jnp.bool_),
        ],
    )(start_pos, end_pos)

    se = jnp.zeros((16,), jnp.int32)
    se = se.at[0:4].set(start_pos).at[4:8].set(end_pos - start_pos)
    tp = _make_tp_sc()(se)
    return (cm, tm, tp)


# speedup vs baseline: 26.6450x; 1.0003x over previous
import functools

import jax
import jax.numpy as jnp
from jax import lax
from jax.experimental import pallas as pl
from jax.experimental.pallas import tpu as pltpu
from jax.experimental.pallas import tpu_sc as plsc

_B = 4
_T = 16
_FRAME = 224 * 224 * 3
_SEQ = _T * _FRAME
_CHUNK = 114688
_NCHUNK = _SEQ // _CHUNK
_VC = _FRAME // 4


def _masks_body(start_ref, end_ref, cm_ref, tm_ref):
    c = pl.program_id(0)
    idx = c * _CHUNK + lax.broadcasted_iota(jnp.int32, (_B, _CHUNK), 1)
    row = lax.broadcasted_iota(jnp.int32, (_B, _CHUNK), 0)

    def per_row(vals_ref):
        v0, v1, v2, v3 = vals_ref[0], vals_ref[1], vals_ref[2], vals_ref[3]
        return jnp.where(row == 0, v0,
               jnp.where(row == 1, v1,
               jnp.where(row == 2, v2, v3)))

    s = per_row(start_ref)
    e = per_row(end_ref)
    tm = (idx >= s) & (idx < e)
    tm_ref[...] = tm
    cm_ref[...] = ~tm


@functools.cache
def _make_tp_sc():
    mesh = plsc.VectorSubcoreMesh(core_axis_name="c", subcore_axis_name="s")
    return functools.partial(
        pl.kernel,
        mesh=mesh,
        out_type=jax.ShapeDtypeStruct((_B, _SEQ), jnp.int32),
        scratch_types=[
            pltpu.VMEM((16,), jnp.int32),
            pltpu.VMEM((_VC,), jnp.int32),
            pltpu.VMEM((_VC,), jnp.int32),
            pltpu.SemaphoreType.DMA,
            pltpu.SemaphoreType.DMA,
        ],
    )(_tp_sc_body)


def _tp_sc_body(se_hbm, tp_out, se_v, buf0, buf1, sem0, sem1):
    wid = lax.axis_index("s") * 2 + lax.axis_index("c")
    pltpu.sync_copy(se_hbm, se_v)
    sev = se_v[...]
    sps = [sev[b] for b in range(_B)]
    ls = [sev[_B + b] for b in range(_B)]

    def sel(vals, b):
        return jnp.where(b == 0, vals[0],
               jnp.where(b == 1, vals[1],
               jnp.where(b == 2, vals[2], vals[3])))

    b = wid // 8
    r8 = wid % 8
    sp = sel(sps, b)
    ln = sel(ls, b)
    lane16 = lax.iota(jnp.int32, 16)

    col0 = r8 * (2 * _FRAME)
    bufs = (buf0, buf1)
    sems = (sem0, sem1)
    copies = [None, None]
    for k in range(8):
        base = col0 + k * _VC
        buf = bufs[k % 2]
        if copies[k % 2] is not None:
            copies[k % 2].wait()

        @pl.when(base < ln)
        def _(base=base, buf=buf):
            def gen(i, v):
                for u in range(8):
                    buf[pl.ds(i * 128 + u * 16, 16)] = v + u * 16
                return v + 128
            lax.fori_loop(0, _VC // 128, gen, sp + base + lane16)

        @pl.when(base >= ln)
        def _(buf=buf):
            cv = jnp.full((16,), _SEQ, jnp.int32)

            def gen(i, v):
                for u in range(8):
                    buf[pl.ds(i * 128 + u * 16, 16)] = v
                return v
            lax.fori_loop(0, _VC // 128, gen, cv)

        copies[k % 2] = pltpu.async_copy(
            buf, tp_out.at[b, pl.ds(base, _VC)], sems[k % 2])
    copies[0].wait()
    copies[1].wait()


def kernel(batch_size, num_frames, frame_size, scales, rand_start):
    num_mask = jnp.clip((scales * _T).astype(jnp.int32), 1, _T - 2)
    max_start = jnp.clip(_T - num_mask - 1, 1, None)
    start_frames = (rand_start * max_start.astype(jnp.float32) + 1.0).astype(jnp.int32)
    start_pos = start_frames * _FRAME
    end_pos = jnp.minimum((start_frames + num_mask) * _FRAME, _SEQ)

    cm, tm = pl.pallas_call(
        _masks_body,
        grid=(_NCHUNK,),
        in_specs=[
            pl.BlockSpec(memory_space=pltpu.SMEM),
            pl.BlockSpec(memory_space=pltpu.SMEM),
        ],
        out_specs=[
            pl.BlockSpec((_B, _CHUNK), lambda c: (0, c)),
            pl.BlockSpec((_B, _CHUNK), lambda c: (0, c)),
        ],
        out_shape=[
            jax.ShapeDtypeStruct((_B, _SEQ), jnp.bool_),
            jax.ShapeDtypeStruct((_B, _SEQ), jnp.bool_),
        ],
    )(start_pos, end_pos)

    se = jnp.zeros((16,), jnp.int32)
    se = se.at[0:4].set(start_pos).at[4:8].set(end_pos - start_pos)
    tp = _make_tp_sc()(se)
    return (cm, tm, tp)


# speedup vs baseline: 26.7028x; 1.0022x over previous
import functools

import jax
import jax.numpy as jnp
from jax import lax
from jax.experimental import pallas as pl
from jax.experimental.pallas import tpu as pltpu
from jax.experimental.pallas import tpu_sc as plsc

_B = 4
_T = 16
_FRAME = 224 * 224 * 3
_SEQ = _T * _FRAME
_CHUNK = 114688
_NCHUNK = _SEQ // _CHUNK
_VC = _FRAME // 4


def _masks_body(start_ref, end_ref, cm_ref, tm_ref):
    c = pl.program_id(0)
    idx = c * _CHUNK + lax.broadcasted_iota(jnp.int32, (_B, _CHUNK), 1)
    row = lax.broadcasted_iota(jnp.int32, (_B, _CHUNK), 0)

    def per_row(vals_ref):
        v0, v1, v2, v3 = vals_ref[0], vals_ref[1], vals_ref[2], vals_ref[3]
        return jnp.where(row == 0, v0,
               jnp.where(row == 1, v1,
               jnp.where(row == 2, v2, v3)))

    s = per_row(start_ref)
    e = per_row(end_ref)
    tm = (idx >= s) & (idx < e)
    tm_ref[...] = tm
    cm_ref[...] = ~tm


@functools.cache
def _make_tp_sc():
    mesh = plsc.VectorSubcoreMesh(core_axis_name="c", subcore_axis_name="s")
    return functools.partial(
        pl.kernel,
        mesh=mesh,
        out_type=jax.ShapeDtypeStruct((_B, _SEQ), jnp.int32),
        scratch_types=[
            pltpu.VMEM((16,), jnp.int32),
            pltpu.VMEM((_VC,), jnp.int32),
            pltpu.VMEM((_VC,), jnp.int32),
            pltpu.SemaphoreType.DMA,
            pltpu.SemaphoreType.DMA,
        ],
    )(_tp_sc_body)


def _tp_sc_body(se_hbm, tp_out, se_v, buf0, buf1, sem0, sem1):
    wid = lax.axis_index("s") * 2 + lax.axis_index("c")
    pltpu.sync_copy(se_hbm, se_v)
    sev = se_v[...]
    sps = [sev[b] for b in range(_B)]
    ls = [sev[_B + b] for b in range(_B)]

    def sel(vals, b):
        return jnp.where(b == 0, vals[0],
               jnp.where(b == 1, vals[1],
               jnp.where(b == 2, vals[2], vals[3])))

    b = wid // 8
    r8 = wid % 8
    sp = sel(sps, b)
    ln = sel(ls, b)
    lane16 = lax.iota(jnp.int32, 16)

    col0 = r8 * (2 * _FRAME)
    bufs = (buf0, buf1)
    sems = (sem0, sem1)
    copies = [None, None]
    for k in range(8):
        base = col0 + k * _VC
        buf = bufs[k % 2]
        if copies[k % 2] is not None:
            copies[k % 2].wait()

        @pl.when(base < ln)
        def _(base=base, buf=buf):
            def gen(i, v):
                for u in range(8):
                    buf[pl.ds(i * 128 + u * 16, 16)] = v + u * 16
                return v + 128
            lax.fori_loop(0, _VC // 128, gen, sp + base + lane16)

        @pl.when(base >= ln)
        def _(buf=buf):
            cv = jnp.full((16,), _SEQ, jnp.int32)

            def gen(i, v):
                for u in range(8):
                    buf[pl.ds(i * 128 + u * 16, 16)] = v
                return v
            lax.fori_loop(0, _VC // 128, gen, cv)

        copies[k % 2] = pltpu.async_copy(
            buf, tp_out.at[b, pl.ds(base, _VC)], sems[k % 2])
    copies[0].wait()
    copies[1].wait()


def kernel(batch_size, num_frames, frame_size, scales, rand_start):
    num_mask = jnp.clip((scales * _T).astype(jnp.int32), 1, _T - 2)
    max_start = jnp.clip(_T - num_mask - 1, 1, None)
    start_frames = (rand_start * max_start.astype(jnp.float32) + 1.0).astype(jnp.int32)
    start_pos = start_frames * _FRAME
    end_pos = jnp.minimum((start_frames + num_mask) * _FRAME, _SEQ)

    se = jnp.zeros((16,), jnp.int32)
    se = se.at[0:4].set(start_pos).at[4:8].set(end_pos - start_pos)
    tp = _make_tp_sc()(se)

    cm, tm = pl.pallas_call(
        _masks_body,
        grid=(_NCHUNK,),
        in_specs=[
            pl.BlockSpec(memory_space=pltpu.SMEM),
            pl.BlockSpec(memory_space=pltpu.SMEM),
        ],
        out_specs=[
            pl.BlockSpec((_B, _CHUNK), lambda c: (0, c)),
            pl.BlockSpec((_B, _CHUNK), lambda c: (0, c)),
        ],
        out_shape=[
            jax.ShapeDtypeStruct((_B, _SEQ), jnp.bool_),
            jax.ShapeDtypeStruct((_B, _SEQ), jnp.bool_),
        ],
    )(start_pos, end_pos)
    return (cm, tm, tp)
